# baseline (speedup 1.0000x reference)
import functools

import jax
import jax.numpy as jnp
from jax.experimental import pallas as pl


def _pe_add_kernel(T, x_ref, emb_ref, o_ref):
    o_ref[...] = x_ref[...] + emb_ref[...][None, :, :]


def _pe_add2d_kernel(x_ref, emb_ref, o_ref):
    o_ref[...] = x_ref[...] + emb_ref[...]


def kernel(x, embedding_weight):
    B, T, D = x.shape
    x2 = x.reshape(B, T * D)
    BBLK = 64
    grid = (B // BBLK,)
    out = pl.pallas_call(
        _pe_add2d_kernel,
        grid=grid,
        in_specs=[
            pl.BlockSpec((BBLK, T * D), lambda i: (i, 0)),
            pl.BlockSpec((1, T * D), lambda i: (0, 0)),
        ],
        out_specs=pl.BlockSpec((BBLK, T * D), lambda i: (i, 0)),
        out_shape=jax.ShapeDtypeStruct((B, T * D), x.dtype),
    )(x2, embedding_weight[:T].reshape(1, T * D))
    return out.reshape(B, T, D)


# speedup vs baseline: 3.5784x; 3.5784x over previous
import functools

import jax
import jax.numpy as jnp
from jax.experimental import pallas as pl


def _pe_add_kernel(T, x_ref, emb_ref, o_ref):
    o_ref[...] = x_ref[...] + emb_ref[...][None, :, :]


def _pe_add2d_kernel(x_ref, emb_ref, o_ref):
    o_ref[...] = x_ref[...] + emb_ref[...]


def kernel(x, embedding_weight):
    B, T, D = x.shape
    BBLK = 64
    grid = (B // BBLK,)
    return pl.pallas_call(
        functools.partial(_pe_add_kernel, T),
        grid=grid,
        in_specs=[
            pl.BlockSpec((BBLK, T, D), lambda i: (i, 0, 0)),
            pl.BlockSpec((T, D), lambda i: (0, 0)),
        ],
        out_specs=pl.BlockSpec((BBLK, T, D), lambda i: (i, 0, 0)),
        out_shape=jax.ShapeDtypeStruct((B, T, D), x.dtype),
    )(x, embedding_weight)


# speedup vs baseline: 3.6375x; 1.0165x over previous
import functools

import jax
import jax.numpy as jnp
from jax.experimental import pallas as pl


def _pe_add_kernel(T, x_ref, emb_ref, o_ref):
    o_ref[...] = x_ref[...] + emb_ref[...][None, :, :]


def _pe_add2d_kernel(x_ref, emb_ref, o_ref):
    o_ref[...] = x_ref[...] + emb_ref[...]


def kernel(x, embedding_weight):
    B, T, D = x.shape
    BBLK = 128
    grid = (B // BBLK,)
    return pl.pallas_call(
        functools.partial(_pe_add_kernel, T),
        grid=grid,
        in_specs=[
            pl.BlockSpec((BBLK, T, D), lambda i: (i, 0, 0)),
            pl.BlockSpec((T, D), lambda i: (0, 0)),
        ],
        out_specs=pl.BlockSpec((BBLK, T, D), lambda i: (i, 0, 0)),
        out_shape=jax.ShapeDtypeStruct((B, T, D), x.dtype),
    )(x, embedding_weight)
